# Initial kernel scaffold; baseline (speedup 1.0000x reference)
import jax
import jax.numpy as jnp
from jax.experimental import pallas as pl


def kernel(conf_matrix, conf_matrix_gt, topic_matrix, spv_b_ids, spv_i_ids, spv_j_ids, expec_f, expec_f_gt):
    raise NotImplementedError("write your pallas kernel here")


# speedup vs baseline: 6.9480x; 6.9480x over previous
import functools

import jax
import jax.numpy as jnp
from jax import lax
from jax.experimental import pallas as pl
from jax.experimental.pallas import tpu as pltpu
from jax.experimental.pallas import tpu_sc as plsc

_ALPHA = 0.25
_EPS = 1e-6
_N, _HW0, _HW1 = 2, 3600, 3600
_M = 5000
_RATIO = 10
_TOTAL = _N * _HW0 * _HW1
_PAD_SLOT = _TOTAL
_TABLE = _TOTAL + 128
_M10 = _M * _RATIO
_NC, _NS = 2, 16
_NW = _NC * _NS
_CHUNK = 128
_CHUNKS = 13
_PER_TILE = _CHUNKS * _CHUNK
_MPAD = _NW * _PER_TILE
_ROWS = _N * _HW0
_BLK = 120
_GRID = _ROWS // _BLK
_FPAD = 5120

def _wid():
    return lax.axis_index("s") * _NC + lax.axis_index("c")


@functools.lru_cache(maxsize=None)
def _sc_kernels():
    mesh = plsc.VectorSubcoreMesh(core_axis_name="c", subcore_axis_name="s",
                                  num_cores=_NC, num_subcores=_NS)

    @functools.partial(
        pl.kernel,
        out_type=jax.ShapeDtypeStruct((_TABLE,), jnp.int32),
        mesh=mesh,
        scratch_types=[
            pltpu.VMEM((_CHUNKS, _CHUNK), jnp.int32),
            pltpu.VMEM((_CHUNKS, _CHUNK), jnp.int32),
            pltpu.SemaphoreType.DMA,
        ],
    )
    def _sc_scatter(idx_hbm, ids_hbm, table_hbm, idx_v, ids_v, sem):
        w = _wid()
        pltpu.sync_copy(idx_hbm.at[w], idx_v)
        pltpu.sync_copy(ids_hbm.at[w], ids_v)
        cps = [
            pltpu.async_copy(ids_v.at[j], table_hbm.at[idx_v.at[j]], sem)
            for j in range(_CHUNKS)
        ]
        for cp in cps:
            cp.wait()

    @functools.partial(
        pl.kernel,
        out_type=(
            jax.ShapeDtypeStruct((_NW, _CHUNKS, _CHUNK), jnp.int32),
            jax.ShapeDtypeStruct((_NW, _CHUNKS, _CHUNK), jnp.float32),
        ),
        mesh=mesh,
        scratch_types=[
            pltpu.VMEM((_CHUNKS, _CHUNK), jnp.int32),
            pltpu.VMEM((_CHUNKS, _CHUNK), jnp.int32),
            pltpu.VMEM((_CHUNKS, _CHUNK), jnp.int32),
            pltpu.VMEM((_CHUNKS, _CHUNK), jnp.float32),
            pltpu.SemaphoreType.DMA,
            pltpu.SemaphoreType.DMA,
        ],
    )
    def _sc_gather(table_hbm, topic_hbm, idx_hbm, idxc_hbm, got_out, val_out,
                   idx_v, idxc_v, got_v, val_v, sem_a, sem_b):
        w = _wid()
        pltpu.sync_copy(idx_hbm.at[w], idx_v)
        pltpu.sync_copy(idxc_hbm.at[w], idxc_v)
        cps = []
        for j in range(_CHUNKS):
            cps.append(
                pltpu.async_copy(table_hbm.at[idx_v.at[j]], got_v.at[j], sem_a))
            cps.append(
                pltpu.async_copy(topic_hbm.at[idxc_v.at[j]], val_v.at[j], sem_b))
        for cp in cps:
            cp.wait()
        pltpu.sync_copy(got_v, got_out.at[w])
        pltpu.sync_copy(val_v, val_out.at[w])

    return _sc_scatter, _sc_gather


def _tc_body(gt_ref, tp_ref, cf_ref, negg_ref, negv_ref,
             e0_ref, e1_ref, sd_ref, g0_ref, g1_ref, out_ref, acc):
    step = pl.program_id(0)

    @pl.when(step == 0)
    def _prologue():
        ids = (lax.broadcasted_iota(jnp.int32, (_MPAD // 128, 128), 0) * 128
               + lax.broadcasted_iota(jnp.int32, (_MPAD // 128, 128), 1))
        win = (negg_ref[...] == ids) & (ids < _M10)
        v = negv_ref[...]
        acc[3] = jnp.sum(jnp.where(win, jnp.log(1.0 - v + _EPS), 0.0))
        acc[4] = jnp.sum(win.astype(jnp.float32))
        fid = (lax.broadcasted_iota(jnp.int32, (_FPAD // 128, 128), 0) * 128
               + lax.broadcasted_iota(jnp.int32, (_FPAD // 128, 128), 1))
        valid = fid < _M
        inv_std = jnp.where(valid, 1.0 / jnp.maximum(sd_ref[...], 1e-10), 0.0)
        acc[7] = jnp.sum(inv_std)
        d0 = g0_ref[...] - e0_ref[...]
        d1 = g1_ref[...] - e1_ref[...]
        off = d0 * d0 + d1 * d1
        corr = valid & (jnp.maximum(jnp.abs(g0_ref[...]),
                                    jnp.abs(g1_ref[...])) < 1.0)
        acc[5] = jnp.sum(jnp.where(corr, off * inv_std, 0.0))
        acc[6] = jnp.sum(corr.astype(jnp.float32))
        acc[0] = 0.0
        acc[1] = 0.0
        acc[2] = 0.0

    posf = (gt_ref[...] == 1).astype(jnp.float32)
    acc[0] += jnp.sum(jnp.log(tp_ref[...] + _EPS) * posf)
    cfv = jnp.clip(cf_ref[...], 1e-6, 1.0 - 1e-6)
    acc[1] += jnp.sum(jnp.log(cfv) * posf)
    acc[2] += jnp.sum(posf)

    @pl.when(step == _GRID - 1)
    def _epilogue():
        cp = jnp.maximum(acc[2], 1.0)
        loss_c = (-_ALPHA * (acc[0] / cp)
                  - _ALPHA * (acc[3] / jnp.maximum(acc[4], 1.0))
                  - _ALPHA * (acc[1] / cp))
        mean_inv = acc[7] / float(_M)
        loss_f = (acc[5] / mean_inv) / jnp.maximum(acc[6], 1.0)
        out_ref[0] = loss_c + loss_f


_tc_in_specs = [
    pl.BlockSpec((_BLK, _HW1), lambda i: (i, 0)),
    pl.BlockSpec((_BLK, _HW1), lambda i: (i, 0)),
    pl.BlockSpec((_BLK, _HW1), lambda i: (i, 0)),
    pl.BlockSpec((_MPAD // 128, 128), lambda i: (0, 0)),
    pl.BlockSpec((_MPAD // 128, 128), lambda i: (0, 0)),
] + [pl.BlockSpec((_FPAD // 128, 128), lambda i: (0, 0))] * 5


def kernel(conf_matrix, conf_matrix_gt, topic_matrix, spv_b_ids, spv_i_ids,
           spv_j_ids, expec_f, expec_f_gt):
    hi = (_HW1 - 1) // 3
    nkey = jax.random.key(42)
    js = []
    for r in range(_RATIO):
        d = jax.random.randint(jax.random.fold_in(nkey, r), (_M,), 0, hi,
                               dtype=spv_j_ids.dtype)
        js.append((spv_j_ids + d * 3 + 1) % _HW1)
    sj = jnp.concatenate(js)
    b10 = jnp.tile(spv_b_ids, _RATIO)
    i10 = jnp.tile(spv_i_ids, _RATIO)
    flat = b10 * (_HW0 * _HW1) + i10 * _HW1 + sj
    pad_tab = jnp.full((_MPAD - _M10,), _PAD_SLOT, jnp.int32)
    pad_top = jnp.zeros((_MPAD - _M10,), jnp.int32)
    idx_tab3 = jnp.concatenate([flat, pad_tab]).reshape(_NW, _CHUNKS, _CHUNK)
    idx_top3 = jnp.concatenate([flat, pad_top]).reshape(_NW, _CHUNKS, _CHUNK)
    ids3 = jnp.arange(_MPAD, dtype=jnp.int32).reshape(_NW, _CHUNKS, _CHUNK)

    sc_scatter, sc_gather = _sc_kernels()
    table = sc_scatter(idx_tab3, ids3)
    topic_flat = topic_matrix.reshape(_TOTAL)
    got3, val3 = sc_gather(table, topic_flat, idx_tab3, idx_top3)

    gt2 = conf_matrix_gt.reshape(_ROWS, _HW1)
    tp2 = topic_matrix.reshape(_ROWS, _HW1)
    cf2 = conf_matrix.reshape(_ROWS, _HW1)
    got2 = got3.reshape(_MPAD // 128, 128)
    val2 = val3.reshape(_MPAD // 128, 128)

    def _pad_col(col):
        return (jnp.concatenate([col, jnp.zeros((_FPAD - _M,), jnp.float32)])
                .reshape(_FPAD // 128, 128))

    e0 = _pad_col(expec_f[:, 0])
    e1 = _pad_col(expec_f[:, 1])
    sd = _pad_col(expec_f[:, 2])
    g0 = _pad_col(expec_f_gt[:, 0])
    g1 = _pad_col(expec_f_gt[:, 1])

    parts = pl.pallas_call(
        _tc_body,
        grid=(_GRID,),
        in_specs=_tc_in_specs,
        out_specs=pl.BlockSpec(memory_space=pltpu.SMEM),
        out_shape=jax.ShapeDtypeStruct((1,), jnp.float32),
        scratch_shapes=[pltpu.SMEM((8,), jnp.float32)],
        compiler_params=pltpu.CompilerParams(
            dimension_semantics=("arbitrary",)),
    )(gt2, tp2, cf2, got2, val2, e0, e1, sd, g0, g1)
    return parts[0]


# speedup vs baseline: 7.0974x; 1.0215x over previous
import functools

import jax
import jax.numpy as jnp
from jax import lax
from jax.experimental import pallas as pl
from jax.experimental.pallas import tpu as pltpu
from jax.experimental.pallas import tpu_sc as plsc

_ALPHA = 0.25
_EPS = 1e-6
_N, _HW0, _HW1 = 2, 3600, 3600
_M = 5000
_RATIO = 10
_TOTAL = _N * _HW0 * _HW1
_PAD_SLOT = _TOTAL
_TABLE = _TOTAL + 128
_M10 = _M * _RATIO
_NC, _NS = 2, 16
_NW = _NC * _NS
_CHUNK = 128
_CHUNKS = 13
_PER_TILE = _CHUNKS * _CHUNK
_MPAD = _NW * _PER_TILE
_ROWS = _N * _HW0
_BLK = 360
_GRID = _ROWS // _BLK
_FPAD = 5120


def _wid():
    return lax.axis_index("s") * _NC + lax.axis_index("c")


@functools.lru_cache(maxsize=None)
def _sc_kernels():
    mesh = plsc.VectorSubcoreMesh(core_axis_name="c", subcore_axis_name="s",
                                  num_cores=_NC, num_subcores=_NS)

    @functools.partial(
        pl.kernel,
        out_type=jax.ShapeDtypeStruct((_TABLE,), jnp.int32),
        mesh=mesh,
        scratch_types=[
            pltpu.VMEM((_CHUNKS, _CHUNK), jnp.int32),
            pltpu.VMEM((_CHUNKS, _CHUNK), jnp.int32),
            pltpu.SemaphoreType.DMA,
        ],
    )
    def _sc_scatter(idx_hbm, ids_hbm, table_hbm, idx_v, ids_v, sem):
        w = _wid()
        pltpu.sync_copy(idx_hbm.at[w], idx_v)
        pltpu.sync_copy(ids_hbm.at[w], ids_v)
        cps = [
            pltpu.async_copy(ids_v.at[j], table_hbm.at[idx_v.at[j]], sem)
            for j in range(_CHUNKS)
        ]
        for cp in cps:
            cp.wait()

    @functools.partial(
        pl.kernel,
        out_type=(
            jax.ShapeDtypeStruct((_NW, _CHUNKS, _CHUNK), jnp.int32),
            jax.ShapeDtypeStruct((_NW, _CHUNKS, _CHUNK), jnp.float32),
        ),
        mesh=mesh,
        scratch_types=[
            pltpu.VMEM((_CHUNKS, _CHUNK), jnp.int32),
            pltpu.VMEM((_CHUNKS, _CHUNK), jnp.int32),
            pltpu.VMEM((_CHUNKS, _CHUNK), jnp.int32),
            pltpu.VMEM((_CHUNKS, _CHUNK), jnp.float32),
            pltpu.SemaphoreType.DMA,
            pltpu.SemaphoreType.DMA,
        ],
    )
    def _sc_gather(table_hbm, topic_hbm, idx_hbm, idxc_hbm, got_out, val_out,
                   idx_v, idxc_v, got_v, val_v, sem_a, sem_b):
        w = _wid()
        pltpu.sync_copy(idx_hbm.at[w], idx_v)
        pltpu.sync_copy(idxc_hbm.at[w], idxc_v)
        cps = []
        for j in range(_CHUNKS):
            cps.append(
                pltpu.async_copy(table_hbm.at[idx_v.at[j]], got_v.at[j], sem_a))
            cps.append(
                pltpu.async_copy(topic_hbm.at[idxc_v.at[j]], val_v.at[j], sem_b))
        for cp in cps:
            cp.wait()
        pltpu.sync_copy(got_v, got_out.at[w])
        pltpu.sync_copy(val_v, val_out.at[w])

    return _sc_scatter, _sc_gather


def _tc_dense_body(gt_ref, tp_ref, cf_ref, out_ref, acc):
    step = pl.program_id(0)

    @pl.when(step == 0)
    def _init():
        acc[0] = 0.0
        acc[1] = 0.0
        acc[2] = 0.0

    posf = (gt_ref[...] == 1).astype(jnp.float32)
    acc[0] += jnp.sum(jnp.log(tp_ref[...] + _EPS) * posf)
    cfv = jnp.clip(cf_ref[...], 1e-6, 1.0 - 1e-6)
    acc[1] += jnp.sum(jnp.log(cfv) * posf)
    acc[2] += jnp.sum(posf)

    @pl.when(step == _GRID - 1)
    def _fin():
        out_ref[0] = acc[0]
        out_ref[1] = acc[1]
        out_ref[2] = acc[2]


def _tc_combine_body(sums_ref, negg_ref, negv_ref,
                     e0_ref, e1_ref, sd_ref, g0_ref, g1_ref, out_ref):
    ids = (lax.broadcasted_iota(jnp.int32, (_MPAD // 128, 128), 0) * 128
           + lax.broadcasted_iota(jnp.int32, (_MPAD // 128, 128), 1))
    win = (negg_ref[...] == ids) & (ids < _M10)
    v = negv_ref[...]
    s_neg = jnp.sum(jnp.where(win, jnp.log(1.0 - v + _EPS), 0.0))
    c_neg = jnp.sum(win.astype(jnp.float32))
    fid = (lax.broadcasted_iota(jnp.int32, (_FPAD // 128, 128), 0) * 128
           + lax.broadcasted_iota(jnp.int32, (_FPAD // 128, 128), 1))
    valid = fid < _M
    inv_std = jnp.where(valid, 1.0 / jnp.maximum(sd_ref[...], 1e-10), 0.0)
    sum_inv = jnp.sum(inv_std)
    d0 = g0_ref[...] - e0_ref[...]
    d1 = g1_ref[...] - e1_ref[...]
    off = d0 * d0 + d1 * d1
    corr = valid & (jnp.maximum(jnp.abs(g0_ref[...]),
                                jnp.abs(g1_ref[...])) < 1.0)
    s_l2 = jnp.sum(jnp.where(corr, off * inv_std, 0.0))
    c_corr = jnp.sum(corr.astype(jnp.float32))

    cp = jnp.maximum(sums_ref[2], 1.0)
    loss_c = (-_ALPHA * (sums_ref[0] / cp)
              - _ALPHA * (s_neg / jnp.maximum(c_neg, 1.0))
              - _ALPHA * (sums_ref[1] / cp))
    mean_inv = sum_inv / float(_M)
    loss_f = (s_l2 / mean_inv) / jnp.maximum(c_corr, 1.0)
    out_ref[0] = loss_c + loss_f


_tc_dense_specs = [
    pl.BlockSpec((_BLK, _HW1), lambda i: (i, 0)),
    pl.BlockSpec((_BLK, _HW1), lambda i: (i, 0)),
    pl.BlockSpec((_BLK, _HW1), lambda i: (i, 0)),
]

_tc_combine_specs = [
    pl.BlockSpec(memory_space=pltpu.SMEM),
    pl.BlockSpec((_MPAD // 128, 128), lambda: (0, 0)),
    pl.BlockSpec((_MPAD // 128, 128), lambda: (0, 0)),
] + [pl.BlockSpec((_FPAD // 128, 128), lambda: (0, 0))] * 5


def kernel(conf_matrix, conf_matrix_gt, topic_matrix, spv_b_ids, spv_i_ids,
           spv_j_ids, expec_f, expec_f_gt):
    hi = (_HW1 - 1) // 3
    nkey = jax.random.key(42)
    js = []
    for r in range(_RATIO):
        d = jax.random.randint(jax.random.fold_in(nkey, r), (_M,), 0, hi,
                               dtype=spv_j_ids.dtype)
        js.append((spv_j_ids + d * 3 + 1) % _HW1)
    sj = jnp.concatenate(js)
    b10 = jnp.tile(spv_b_ids, _RATIO)
    i10 = jnp.tile(spv_i_ids, _RATIO)
    flat = b10 * (_HW0 * _HW1) + i10 * _HW1 + sj
    pad_tab = jnp.full((_MPAD - _M10,), _PAD_SLOT, jnp.int32)
    pad_top = jnp.zeros((_MPAD - _M10,), jnp.int32)
    idx_tab3 = jnp.concatenate([flat, pad_tab]).reshape(_NW, _CHUNKS, _CHUNK)
    idx_top3 = jnp.concatenate([flat, pad_top]).reshape(_NW, _CHUNKS, _CHUNK)
    ids3 = jnp.arange(_MPAD, dtype=jnp.int32).reshape(_NW, _CHUNKS, _CHUNK)

    sc_scatter, sc_gather = _sc_kernels()
    table = sc_scatter(idx_tab3, ids3)
    topic_flat = topic_matrix.reshape(_TOTAL)
    got3, val3 = sc_gather(table, topic_flat, idx_tab3, idx_top3)

    gt2 = conf_matrix_gt.reshape(_ROWS, _HW1)
    tp2 = topic_matrix.reshape(_ROWS, _HW1)
    cf2 = conf_matrix.reshape(_ROWS, _HW1)

    sums = pl.pallas_call(
        _tc_dense_body,
        grid=(_GRID,),
        in_specs=_tc_dense_specs,
        out_specs=pl.BlockSpec(memory_space=pltpu.SMEM),
        out_shape=jax.ShapeDtypeStruct((3,), jnp.float32),
        scratch_shapes=[pltpu.SMEM((3,), jnp.float32)],
        compiler_params=pltpu.CompilerParams(
            dimension_semantics=("arbitrary",)),
    )(gt2, tp2, cf2)

    got2 = got3.reshape(_MPAD // 128, 128)
    val2 = val3.reshape(_MPAD // 128, 128)

    def _pad_col(col):
        return (jnp.concatenate([col, jnp.zeros((_FPAD - _M,), jnp.float32)])
                .reshape(_FPAD // 128, 128))

    e0 = _pad_col(expec_f[:, 0])
    e1 = _pad_col(expec_f[:, 1])
    sd = _pad_col(expec_f[:, 2])
    g0 = _pad_col(expec_f_gt[:, 0])
    g1 = _pad_col(expec_f_gt[:, 1])

    loss = pl.pallas_call(
        _tc_combine_body,
        in_specs=_tc_combine_specs,
        out_specs=pl.BlockSpec(memory_space=pltpu.SMEM),
        out_shape=jax.ShapeDtypeStruct((1,), jnp.float32),
    )(sums, got2, val2, e0, e1, sd, g0, g1)
    return loss[0]
